# baseline (speedup 1.0000x reference)
import functools

import jax
import jax.numpy as jnp
from jax import lax
from jax.experimental import pallas as pl
from jax.experimental.pallas import tpu as pltpu
from jax.experimental.pallas import tpu_sc as plsc

VOCAB = 1000000
EMBED_DIM = 128
CTX = 16384

NC = 2
NS = 16
NW = NC * NS
IDX_PER_W = CTX // NW
CHUNK = 128
NCHUNK = IDX_PER_W // CHUNK
NLANE = EMBED_DIM // 16

_mesh = plsc.VectorSubcoreMesh(core_axis_name="c", subcore_axis_name="s")


@functools.partial(
    pl.kernel,
    mesh=_mesh,
    out_type=jax.ShapeDtypeStruct((NW, EMBED_DIM), jnp.float32),
    scratch_types=[
        pltpu.VMEM((NCHUNK, CHUNK), jnp.int32),
        pltpu.VMEM((NCHUNK, CHUNK, EMBED_DIM), jnp.float32),
        pltpu.VMEM((EMBED_DIM,), jnp.float32),
        pltpu.SemaphoreType.DMA,
    ],
)
def _gather_sum(idx_hbm, table_hbm, out_hbm, idx_v, rows_v, acc_v, sem):
    wid = lax.axis_index("s") * NC + lax.axis_index("c")
    pltpu.sync_copy(idx_hbm.at[wid], idx_v)
    copies = []
    for j in range(NCHUNK):
        copies.append(pltpu.async_copy(table_hbm.at[idx_v.at[j]], rows_v.at[j], sem))
    for c in copies:
        c.wait()

    acc = tuple(jnp.zeros((16,), jnp.float32) for _ in range(NLANE))
    for j in range(NCHUNK):
        def body(r, carry, j=j):
            return tuple(
                carry[c] + rows_v[j, r, c * 16:(c + 1) * 16]
                for c in range(NLANE)
            )
        acc = lax.fori_loop(0, CHUNK, body, acc)
    for c in range(NLANE):
        acc_v[c * 16:(c + 1) * 16] = acc[c]
    pltpu.sync_copy(acc_v, out_hbm.at[wid])


BLOCK_V = 16384
OUT_BLOCK = 2 * BLOCK_V
GRID_V = (VOCAB + OUT_BLOCK - 1) // OUT_BLOCK


def _matvec_kernel(p_ref, w1_ref, w2_ref, b_ref, o_ref):
    hidden = jnp.sum(p_ref[...], axis=0, keepdims=True) * (1.0 / CTX)
    dn = (((1,), (1,)), ((), ()))
    a1 = lax.dot_general(hidden, w1_ref[...], dn, preferred_element_type=jnp.float32)
    a2 = lax.dot_general(hidden, w2_ref[...], dn, preferred_element_type=jnp.float32)
    o_ref[...] = jnp.concatenate([a1, a2], axis=1) + b_ref[...]


_matvec = pl.pallas_call(
    _matvec_kernel,
    grid=(GRID_V,),
    in_specs=[
        pl.BlockSpec((NW, EMBED_DIM), lambda i: (0, 0)),
        pl.BlockSpec((BLOCK_V, EMBED_DIM), lambda i: (2 * i, 0)),
        pl.BlockSpec((BLOCK_V, EMBED_DIM), lambda i: (2 * i + 1, 0)),
        pl.BlockSpec((1, OUT_BLOCK), lambda i: (0, i)),
    ],
    out_specs=pl.BlockSpec((1, OUT_BLOCK), lambda i: (0, i)),
    out_shape=jax.ShapeDtypeStruct((1, VOCAB), jnp.float32),
)


PROBE_CHUNK = 256
PROBE_NCH = 16


@functools.partial(
    pl.kernel,
    mesh=_mesh,
    out_type=jax.ShapeDtypeStruct((NW, EMBED_DIM), jnp.float32),
    scratch_types=[
        pltpu.VMEM((2, PROBE_CHUNK, EMBED_DIM), jnp.float32),
        pltpu.VMEM((EMBED_DIM,), jnp.float32),
        pltpu.SemaphoreType.DMA,
        pltpu.SemaphoreType.DMA,
    ],
)
def _stream_probe(w_hbm, out_hbm, buf, stage, sem0, sem1):
    wid = lax.axis_index("s") * NC + lax.axis_index("c")
    base = wid * (PROBE_NCH * PROBE_CHUNK)
    sems = (sem0, sem1)
    handles = []
    for j in range(PROBE_NCH):
        if j >= 2:
            handles[j - 2].wait()
        handles.append(pltpu.async_copy(
            w_hbm.at[pl.ds(base + j * PROBE_CHUNK, PROBE_CHUNK)],
            buf.at[j % 2], sems[j % 2]))
    handles[-2].wait()
    handles[-1].wait()
    for c in range(NLANE):
        stage[c * 16:(c + 1) * 16] = buf[0, 0, c * 16:(c + 1) * 16] + buf[1, 0, c * 16:(c + 1) * 16]
    pltpu.sync_copy(stage, out_hbm.at[wid])


def kernel(context_indices, in_embeddings, out_weight, out_bias):
    idx3 = context_indices.reshape(NW, NCHUNK, CHUNK)
    partials = _gather_sum(idx3, in_embeddings)
    probe = _stream_probe(out_weight)
    logits = _matvec(partials, out_weight, out_weight,
                     out_bias.reshape(1, VOCAB))
    return logits + probe[0, 0] * 1e-40


# speedup vs baseline: 1.1521x; 1.1521x over previous
import functools

import jax
import jax.numpy as jnp
from jax import lax
from jax.experimental import pallas as pl
from jax.experimental.pallas import tpu as pltpu
from jax.experimental.pallas import tpu_sc as plsc

VOCAB = 1000000
EMBED_DIM = 128
CTX = 16384

NC = 2
NS = 16
NW = NC * NS
IDX_PER_W = CTX // NW
CHUNK = 128
NCHUNK = IDX_PER_W // CHUNK
NLANE = EMBED_DIM // 16

_mesh = plsc.VectorSubcoreMesh(core_axis_name="c", subcore_axis_name="s")


@functools.partial(
    pl.kernel,
    mesh=_mesh,
    out_type=jax.ShapeDtypeStruct((NW, EMBED_DIM), jnp.float32),
    scratch_types=[
        pltpu.VMEM((NCHUNK, CHUNK), jnp.int32),
        pltpu.VMEM((NCHUNK, CHUNK, EMBED_DIM), jnp.float32),
        pltpu.VMEM((EMBED_DIM,), jnp.float32),
        pltpu.SemaphoreType.DMA,
    ],
)
def _gather_sum(idx_hbm, table_hbm, out_hbm, idx_v, rows_v, acc_v, sem):
    wid = lax.axis_index("s") * NC + lax.axis_index("c")
    pltpu.sync_copy(idx_hbm.at[wid], idx_v)
    copies = [
        pltpu.async_copy(table_hbm.at[idx_v.at[j]], rows_v.at[j], sem)
        for j in range(NCHUNK)
    ]
    acc = tuple(jnp.zeros((16,), jnp.float32) for _ in range(NLANE))
    for j in range(NCHUNK):
        copies[j].wait()

        def body(r, carry, j=j):
            return tuple(
                carry[c] + rows_v[j, r, c * 16:(c + 1) * 16]
                for c in range(NLANE)
            )
        acc = lax.fori_loop(0, CHUNK, body, acc)
    for c in range(NLANE):
        acc_v[c * 16:(c + 1) * 16] = acc[c]
    pltpu.sync_copy(acc_v, out_hbm.at[wid])


BLOCK_V = 16384
GRID_V = (VOCAB + BLOCK_V - 1) // BLOCK_V


def _matvec_kernel(p_ref, w_ref, b_ref, o_ref):
    hidden = jnp.sum(p_ref[...], axis=0, keepdims=True) * (1.0 / CTX)
    acc = lax.dot_general(
        hidden, w_ref[...], (((1,), (1,)), ((), ())),
        preferred_element_type=jnp.float32,
    )
    o_ref[...] = acc + b_ref[...]


_matvec = pl.pallas_call(
    _matvec_kernel,
    grid=(GRID_V,),
    in_specs=[
        pl.BlockSpec((NW, EMBED_DIM), lambda i: (0, 0)),
        pl.BlockSpec((BLOCK_V, EMBED_DIM), lambda i: (i, 0)),
        pl.BlockSpec((1, BLOCK_V), lambda i: (0, i)),
    ],
    out_specs=pl.BlockSpec((1, BLOCK_V), lambda i: (0, i)),
    out_shape=jax.ShapeDtypeStruct((1, VOCAB), jnp.float32),
)


def kernel(context_indices, in_embeddings, out_weight, out_bias):
    idx3 = context_indices.reshape(NW, NCHUNK, CHUNK)
    partials = _gather_sum(idx3, in_embeddings)
    return _matvec(partials, out_weight, out_bias.reshape(1, VOCAB))
